# Initial kernel scaffold; baseline (speedup 1.0000x reference)
import functools

import jax
import jax.numpy as jnp
from jax import lax
from jax.experimental import pallas as pl
from jax.experimental.pallas import tpu as pltpu
from jax.experimental.pallas import tpu_sc as plsc

_BATCH = 4096
_HIST = 200
_HALF = 32
_N = _BATCH * _HIST
_NW = 32
_PER_W = _N // _NW
_CH = 128
_NCH = _PER_W // _CH


@functools.partial(
    pl.kernel,
    out_type=jax.ShapeDtypeStruct((_N, 2 * _HALF), jnp.float32),
    mesh=plsc.VectorSubcoreMesh(core_axis_name="c", subcore_axis_name="s"),
    scratch_types=[
        pltpu.VMEM((_CH,), jnp.int32),
        pltpu.VMEM((_CH,), jnp.int32),
        pltpu.VMEM((_CH, _HALF), jnp.float32),
        pltpu.VMEM((_CH, _HALF), jnp.float32),
        pltpu.SemaphoreType.DMA,
    ],
)
def _gather_kernel(ids, item_t, year_t, ylut, out, idx_v, yidx_v, ie_v, ye_v, sem):
    wid = lax.axis_index("s") * 2 + lax.axis_index("c")
    w0 = wid * _PER_W

    def step(i, carry):
        base = w0 + i * _CH
        pltpu.sync_copy(ids.at[pl.ds(base, _CH)], idx_v)
        pltpu.async_copy(ylut.at[idx_v], yidx_v, sem).wait()
        pltpu.async_copy(item_t.at[idx_v], ie_v, sem).wait()
        pltpu.async_copy(year_t.at[yidx_v], ye_v, sem).wait()
        pltpu.sync_copy(ie_v, out.at[pl.ds(base, _CH), pl.ds(0, _HALF)])
        pltpu.sync_copy(ye_v, out.at[pl.ds(base, _CH), pl.ds(_HALF, _HALF)])
        return carry

    lax.fori_loop(0, _NCH, step, 0)


def kernel(item_ids, item_emb, year_emb, year_lookup_table):
    ids = item_ids.reshape(-1)
    out = _gather_kernel(ids, item_emb, year_emb, year_lookup_table)
    return out.reshape(_BATCH, _HIST, 2 * _HALF)


# speedup vs baseline: 1.4364x; 1.4364x over previous
import functools

import jax
import jax.numpy as jnp
from jax import lax
from jax.experimental import pallas as pl
from jax.experimental.pallas import tpu as pltpu
from jax.experimental.pallas import tpu_sc as plsc

_BATCH = 4096
_HIST = 200
_HALF = 32
_N = _BATCH * _HIST
_NW = 32
_PER_W = _N // _NW
_CH = 128
_NCH = _PER_W // _CH


@functools.partial(
    pl.kernel,
    out_type=jax.ShapeDtypeStruct((_N, 2 * _HALF), jnp.float32),
    mesh=plsc.VectorSubcoreMesh(core_axis_name="c", subcore_axis_name="s"),
    scratch_types=[
        pltpu.VMEM((_CH,), jnp.int32),
        pltpu.VMEM((_CH,), jnp.int32),
        pltpu.VMEM((_CH, _HALF), jnp.float32),
        pltpu.VMEM((_CH, _HALF), jnp.float32),
        pltpu.SemaphoreType.DMA,
    ],
    compiler_params=pltpu.CompilerParams(use_tc_tiling_on_sc=False),
)
def _gather_kernel(ids, item_t, year_t, ylut, out, idx_v, yidx_v, ie_v, ye_v, sem):
    wid = lax.axis_index("s") * 2 + lax.axis_index("c")
    w0 = wid * _PER_W

    def step(i, carry):
        base = w0 + i * _CH
        pltpu.sync_copy(ids.at[pl.ds(base, _CH)], idx_v)
        pltpu.async_copy(ylut.at[idx_v], yidx_v, sem).wait()
        pltpu.async_copy(item_t.at[idx_v], ie_v, sem).wait()
        pltpu.async_copy(year_t.at[yidx_v], ye_v, sem).wait()
        pltpu.sync_copy(ie_v, out.at[pl.ds(base, _CH), pl.ds(0, _HALF)])
        pltpu.sync_copy(ye_v, out.at[pl.ds(base, _CH), pl.ds(_HALF, _HALF)])
        return carry

    lax.fori_loop(0, _NCH, step, 0)


def kernel(item_ids, item_emb, year_emb, year_lookup_table):
    ids = item_ids.reshape(-1)
    out = _gather_kernel(ids, item_emb, year_emb, year_lookup_table)
    return out.reshape(_BATCH, _HIST, 2 * _HALF)


# speedup vs baseline: 1.8657x; 1.2989x over previous
import functools

import jax
import jax.numpy as jnp
from jax import lax
from jax.experimental import pallas as pl
from jax.experimental.pallas import tpu as pltpu
from jax.experimental.pallas import tpu_sc as plsc

_BATCH = 4096
_HIST = 200
_HALF = 32
_N = _BATCH * _HIST
_NW = 32
_CH = 128
_GK = 4
_G = _CH * _GK
_ROWS_W = _N // _NW // _CH
_NGRP = _ROWS_W // _GK
_NIT = _NGRP // 2


def _scratch_set():
    return [
        pltpu.VMEM((_GK, _CH), jnp.int32),
        pltpu.VMEM((_GK, _CH), jnp.int32),
        pltpu.VMEM((_G, _HALF), jnp.float32),
        pltpu.VMEM((_G, _HALF), jnp.float32),
        pltpu.SemaphoreType.DMA,
        pltpu.SemaphoreType.DMA,
        pltpu.SemaphoreType.DMA,
        pltpu.SemaphoreType.DMA,
        pltpu.SemaphoreType.DMA,
        pltpu.SemaphoreType.DMA,
    ]


@functools.partial(
    pl.kernel,
    out_type=jax.ShapeDtypeStruct((_N, 2 * _HALF), jnp.float32),
    mesh=plsc.VectorSubcoreMesh(core_axis_name="c", subcore_axis_name="s"),
    scratch_types=_scratch_set() + _scratch_set(),
    compiler_params=pltpu.CompilerParams(use_tc_tiling_on_sc=False),
)
def _gather_kernel(ids, item_t, year_t, ylut, out,
                   idx0, yidx0, ie0, ye0, sa0, sb0, sc0, sd0, se0, sf0,
                   idx1, yidx1, ie1, ye1, sa1, sb1, sc1, sd1, se1, sf1):
    wid = lax.axis_index("s") * 2 + lax.axis_index("c")
    wrow = wid * _ROWS_W

    sets = (
        (idx0, yidx0, ie0, ye0, sa0, sb0, sc0, sd0, se0, sf0),
        (idx1, yidx1, ie1, ye1, sa1, sb1, sc1, sd1, se1, sf1),
    )

    def out_desc(row, bufs):
        idx, yidx, ie, ye, sa, sb, sc, sd, se, sf = bufs
        base = row * _CH
        de = pltpu.make_async_copy(
            ie, out.at[pl.ds(base, _G), pl.ds(0, _HALF)], se)
        df = pltpu.make_async_copy(
            ye, out.at[pl.ds(base, _G), pl.ds(_HALF, _HALF)], sf)
        return de, df

    def run_group(row, bufs):
        idx, yidx, ie, ye, sa, sb, sc, sd, se, sf = bufs
        pltpu.async_copy(ids.at[pl.ds(row, _GK)], idx, sa).wait()
        for b in range(_GK):
            pltpu.async_copy(ylut.at[idx.at[b]], yidx.at[b], sb)
            pltpu.async_copy(item_t.at[idx.at[b]],
                             ie.at[pl.ds(b * _CH, _CH)], sc)
        for b in range(_GK):
            pltpu.make_async_copy(ylut.at[idx.at[b]], yidx.at[b], sb).wait()
            pltpu.async_copy(year_t.at[yidx.at[b]],
                             ye.at[pl.ds(b * _CH, _CH)], sd)
        de, df = out_desc(row, bufs)
        for b in range(_GK):
            pltpu.make_async_copy(item_t.at[idx.at[b]],
                                  ie.at[pl.ds(b * _CH, _CH)], sc).wait()
        de.start()
        for b in range(_GK):
            pltpu.make_async_copy(year_t.at[yidx.at[b]],
                                  ye.at[pl.ds(b * _CH, _CH)], sd).wait()
        df.start()

    def step(i, carry):
        for s in (0, 1):
            row = wrow + (2 * i + s) * _GK

            @pl.when(i > 0)
            def _():
                de, df = out_desc(row - 2 * _GK, sets[s])
                de.wait()
                df.wait()

            run_group(row, sets[s])
        return carry

    lax.fori_loop(0, _NIT, step, 0)

    for s in (0, 1):
        de, df = out_desc(wrow + (2 * (_NIT - 1) + s) * _GK, sets[s])
        de.wait()
        df.wait()


def kernel(item_ids, item_emb, year_emb, year_lookup_table):
    ids = item_ids.reshape(_N // _CH, _CH)
    out = _gather_kernel(ids, item_emb, year_emb, year_lookup_table)
    return out.reshape(_BATCH, _HIST, 2 * _HALF)
